# Initial kernel scaffold; baseline (speedup 1.0000x reference)
import jax
import jax.numpy as jnp
from jax.experimental import pallas as pl


def kernel(pred, y, idx, skip_size):
    raise NotImplementedError("write your pallas kernel here")


# speedup vs baseline: 9.9915x; 9.9915x over previous
import functools

import jax
import jax.numpy as jnp
from jax import lax
from jax.experimental import pallas as pl
from jax.experimental.pallas import tpu as pltpu
from jax.experimental.pallas import tpu_sc as plsc

N = 32768
NUM_SEG = 128
NUM_STATS = 6
L = 16
NC, NS = 2, 16
NW = NC * NS
CHUNK = N // NW
HIST = NUM_SEG * L
RED = NUM_STATS * NUM_SEG


def _sc_body(pred_hbm, y_hbm, dates_hbm, out_hbm,
             pred_v, y_v, dates_v, hist_v, red_v):
    wid = lax.axis_index("c") * NS + lax.axis_index("s")
    base = wid * CHUNK

    pltpu.sync_copy(pred_hbm.at[pl.ds(base, CHUNK)], pred_v)
    pltpu.sync_copy(y_hbm.at[pl.ds(base, CHUNK)], y_v)
    pltpu.sync_copy(dates_hbm.at[pl.ds(base, CHUNK)], dates_v)

    lane = lax.iota(jnp.int32, L)
    zeros = jnp.zeros((L,), jnp.float32)
    ones = jnp.ones((L,), jnp.float32)

    def zero_blk(o, _):
        for u in range(8):
            hist_v[pl.ds((o * 8 + u) * L, L)] = zeros
        return 0
    lax.fori_loop(0, (NUM_STATS * HIST) // (8 * L), zero_blk, 0)

    def accum(o, _):
        for u in range(4):
            j = o * 4 + u
            p = pred_v[pl.ds(j * L, L)]
            t = y_v[pl.ds(j * L, L)]
            d = dates_v[pl.ds(j * L, L)]
            flat = d * L + lane
            plsc.addupdate_scatter(hist_v, [flat], ones)
            plsc.addupdate_scatter(hist_v, [flat + HIST], p)
            plsc.addupdate_scatter(hist_v, [flat + 2 * HIST], t)
            plsc.addupdate_scatter(hist_v, [flat + 3 * HIST], p * p)
            plsc.addupdate_scatter(hist_v, [flat + 4 * HIST], t * t)
            plsc.addupdate_scatter(hist_v, [flat + 5 * HIST], p * t)
        return 0
    lax.fori_loop(0, (CHUNK // L) // 4, accum, 0)

    last = lane == (L - 1)

    def reduce_blk(o, _):
        for u in range(8):
            g = o * 8 + u
            v = hist_v[pl.ds(g * L, L)]
            s = jnp.cumsum(v)
            plsc.store_scatter(red_v, [jnp.full((L,), g, jnp.int32)], s,
                               mask=last)
        return 0
    lax.fori_loop(0, RED // 8, reduce_blk, 0)

    pltpu.sync_copy(red_v, out_hbm.at[wid])


def _sc_hist(pred, y, dates):
    mesh = plsc.VectorSubcoreMesh(core_axis_name="c", subcore_axis_name="s")
    f = pl.kernel(
        _sc_body, mesh=mesh,
        out_type=jax.ShapeDtypeStruct((NW, RED), jnp.float32),
        compiler_params=pltpu.CompilerParams(needs_layout_passes=False),
        scratch_types=[
            pltpu.VMEM((CHUNK,), jnp.float32),
            pltpu.VMEM((CHUNK,), jnp.float32),
            pltpu.VMEM((CHUNK,), jnp.int32),
            pltpu.VMEM((NUM_STATS * HIST,), jnp.float32),
            pltpu.VMEM((RED,), jnp.float32),
        ],
    )
    return f(pred, y, dates)


def _tc_combine_body(part_ref, skip_ref, out_ref):
    EPS = 1e-12
    n = jnp.sum(part_ref[:, 0:128], axis=0, keepdims=True)
    sp = jnp.sum(part_ref[:, 128:256], axis=0, keepdims=True)
    sy = jnp.sum(part_ref[:, 256:384], axis=0, keepdims=True)
    spp = jnp.sum(part_ref[:, 384:512], axis=0, keepdims=True)
    syy = jnp.sum(part_ref[:, 512:640], axis=0, keepdims=True)
    spy = jnp.sum(part_ref[:, 640:768], axis=0, keepdims=True)
    safe_n = jnp.maximum(n, 1.0)
    safe_nm1 = jnp.maximum(n - 1.0, 1.0)
    pm = sp / safe_n
    ym = sy / safe_n
    pvar = jnp.maximum((spp - n * pm * pm) / safe_nm1, 0.0)
    yvar = jnp.maximum((syy - n * ym * ym) / safe_nm1, 0.0)
    pstd = jnp.where(pvar > 0.0, jnp.sqrt(jnp.where(pvar > 0.0, pvar, 1.0)), 0.0)
    ystd = jnp.where(yvar > 0.0, jnp.sqrt(jnp.where(yvar > 0.0, yvar, 1.0)), 0.0)
    cross = spy - n * pm * ym
    valid = (n >= skip_ref[0, 0]) & (pstd >= EPS) & (ystd >= EPS)
    denom = jnp.where(valid, n * pstd * ystd, 1.0)
    ic = jnp.where(valid, cross / denom, 0.0)
    num_valid = jnp.sum(valid.astype(jnp.float32))
    out_ref[:, :] = (-jnp.sum(ic) / num_valid).reshape(1, 1)


def _tc_combine(partials, skip):
    return pl.pallas_call(
        _tc_combine_body,
        out_shape=jax.ShapeDtypeStruct((1, 1), jnp.float32),
    )(partials, skip)


def kernel(pred, y, idx, skip_size):
    dates = idx[:, 0].astype(jnp.int32)
    partials = _sc_hist(pred, y, dates)
    skip = jnp.asarray(skip_size, jnp.float32).reshape(1, 1)
    out = _tc_combine(partials, skip)
    return out[0, 0]
